# baseline (speedup 1.0000x reference)
import functools

import jax
import jax.numpy as jnp
import numpy as np
from jax import lax
from jax.experimental import pallas as pl
from jax.experimental.pallas import tpu as pltpu
from jax.experimental.pallas import tpu_sc as plsc

_NUM_L = 3
_NUM_LM = 9
_AVG_NUM_NEIGHBORS = 16.0
_LM_TO_L = np.repeat(np.arange(_NUM_L), [2 * l + 1 for l in range(_NUM_L)])

_B = 64
_LANES = 16
_CL = 32
_NC = 2
_NS = 16
_NIN = 2
_NSC = 2


def _dense_pre(node_feats, node_attrs, w_up_s, w_skip_flat):
    n, c = node_feats.shape
    ne = node_attrs.shape[1]
    blk = 1000
    grid = n // blk

    def body(nf_ref, na_ref, wup_ref, wsk_ref, h_ref, sc_ref):
        nf = nf_ref[...]
        h_ref[...] = jnp.dot(nf, wup_ref[...], preferred_element_type=jnp.float32)
        t = jnp.dot(nf, wsk_ref[...], preferred_element_type=jnp.float32)
        na = na_ref[...]
        sc_ref[...] = jnp.sum(t.reshape(blk, ne, c) * na[:, :, None], axis=1)

    return pl.pallas_call(
        body,
        grid=(grid,),
        in_specs=[
            pl.BlockSpec((blk, c), lambda i: (i, 0)),
            pl.BlockSpec((blk, ne), lambda i: (i, 0)),
            pl.BlockSpec((c, c), lambda i: (0, 0)),
            pl.BlockSpec((c, ne * c), lambda i: (0, 0)),
        ],
        out_specs=[
            pl.BlockSpec((blk, c), lambda i: (i, 0)),
            pl.BlockSpec((blk, c), lambda i: (i, 0)),
        ],
        out_shape=[
            jax.ShapeDtypeStruct((n, c), jnp.float32),
            jax.ShapeDtypeStruct((n, c), jnp.float32),
        ],
    )(node_feats, node_attrs, w_up_s, w_skip_flat)


def _dense_post(partials, w_msg_e):
    _, nr, c = partials.shape
    n = nr // _NUM_LM
    blk = 1000
    grid = n // blk

    def body(p_ref, w_ref, o_ref):
        msg = (p_ref[0].astype(jnp.float32) +
               p_ref[1].astype(jnp.float32)).reshape(blk, _NUM_LM, c)
        for lm in range(_NUM_LM):
            o_ref[:, lm, :] = jnp.dot(msg[:, lm, :], w_ref[lm],
                                      preferred_element_type=jnp.float32)

    return pl.pallas_call(
        body,
        grid=(grid,),
        in_specs=[
            pl.BlockSpec((2, blk * _NUM_LM, c), lambda i: (0, i, 0)),
            pl.BlockSpec((_NUM_LM, c, c), lambda i: (0, 0, 0)),
        ],
        out_specs=pl.BlockSpec((blk, _NUM_LM, c), lambda i: (i, 0, 0)),
        out_shape=jax.ShapeDtypeStruct((n, _NUM_LM, c), jnp.float32),
    )(partials, w_msg_e)


def _make_sc_kernel(n, e, c):
    k_chunks = c // _CL
    blocks = e // _B
    blocks_per_core = blocks // _NC
    rows_per_tile = n // _NS
    nslots = 80

    mesh = plsc.VectorSubcoreMesh(core_axis_name="c", subcore_axis_name="s")

    scratch = [pltpu.VMEM_SHARED((n, _NUM_LM, _CL), jnp.bfloat16)]
    scratch += [pltpu.VMEM_SHARED((n, _CL), jnp.bfloat16)]
    scratch += [pltpu.VMEM((2 * _B,), jnp.int32) for _ in range(_NIN)]
    scratch += [pltpu.VMEM((_B * _NUM_LM + _LANES,), jnp.float32)
                for _ in range(_NIN)]
    scratch += [pltpu.VMEM((_NUM_L, _B, _CL), jnp.float32)
                for _ in range(_NIN)]
    scratch += [pltpu.VMEM((_B,), jnp.int32) for _ in range(_NSC)]
    scratch += [pltpu.VMEM((_B, _CL), jnp.bfloat16) for _ in range(_NSC)]
    scratch += [pltpu.VMEM((_B // 2,), jnp.int32) for _ in range(_NSC)]
    scratch += [pltpu.VMEM((_B // 2, _NUM_LM, _CL), jnp.bfloat16)
                for _ in range(_NSC)]
    scratch += [pltpu.SemaphoreType.DMA for _ in range(_NIN)]
    scratch += [pltpu.SemaphoreType.DMA for _ in range(_NSC)]
    scratch += [pltpu.SemaphoreType.DMA for _ in range(_NSC)]

    @functools.partial(
        pl.kernel,
        out_type=jax.ShapeDtypeStruct((_NC, n, _NUM_LM, c), jnp.bfloat16),
        mesh=mesh,
        compiler_params=pltpu.CompilerParams(use_tc_tiling_on_sc=False,
                                             needs_layout_passes=False),
        scratch_types=scratch,
    )
    def sc_msg(h_hbm, ef_hbm, ea_hbm, sr_hbm, z_hbm, out_hbm, acc, h_sh,
               *sc):
        srb = sc[0:_NIN]
        av = sc[_NIN:2 * _NIN]
        rv = sc[2 * _NIN:3 * _NIN]
        o = 3 * _NIN
        gidx = sc[o:o + _NSC]
        hv = sc[o + _NSC:o + 2 * _NSC]
        ridx_s = sc[o + 2 * _NSC:o + 3 * _NSC]
        mv = sc[o + 3 * _NSC:o + 4 * _NSC]
        o = o + 4 * _NSC
        sem_in = sc[o:o + _NIN]
        sem_g = sc[o + _NIN:o + _NIN + _NSC]
        sem_sc = sc[o + _NIN + _NSC:o + _NIN + 2 * _NSC]

        cid = lax.axis_index("c")
        sid = lax.axis_index("s")
        row0 = sid * rows_per_tile

        def blk_of(m):
            return cid * blocks_per_core + sid + _NS * m

        def base_of(m):
            return blk_of(m) * _B

        def valid(m):
            return sid + _NS * m < blocks_per_core

        def in_descs(m, bi, k):
            b = base_of(m)
            return [
                (sr_hbm.at[blk_of(m), :], srb[bi]),
                (ea_hbm.at[pl.ds(b * _NUM_LM, _B * _NUM_LM)],
                 av[bi].at[pl.ds(0, _B * _NUM_LM)]),
            ] + [
                (ef_hbm.at[pl.ds(b, _B),
                           pl.ds(l * (k_chunks * _CL) + k * _CL, _CL)],
                 rv[bi].at[l])
                for l in range(_NUM_L)
            ]

        def issue_in(m, bi, k):
            @pl.when(valid(m))
            def _():
                for s, d in in_descs(m, bi, k):
                    pltpu.async_copy(s, d, sem_in[bi])

        def wait_in(m, bi, k):
            @pl.when(valid(m))
            def _():
                for s, d in in_descs(m, bi, k):
                    pltpu.make_async_copy(s, d, sem_in[bi]).wait()

        def issue_gather(m, bi, gi, k):
            @pl.when(valid(m))
            def _():
                for i in range(_B // _LANES):
                    gidx[gi][pl.ds(i * _LANES, _LANES)] = (
                        srb[bi][pl.ds(i * _LANES, _LANES)])
                pltpu.async_copy(h_sh.at[gidx[gi]], hv[gi], sem_g[gi])

        def wait_gather(m, gi):
            @pl.when(valid(m))
            def _():
                pltpu.make_async_copy(h_sh.at[gidx[gi]], hv[gi],
                                      sem_g[gi]).wait()

        def wait_scatter(m, half):
            @pl.when(valid(m) & (m >= 0))
            def _():
                pltpu.make_async_copy(mv[half], acc.at[ridx_s[half]],
                                      sem_sc[half]).wait()

        def compute_and_scatter(m, bi, gi):
            hb = _B // 2
            for half in range(2):
                wait_scatter(m - 1, half)

                @pl.when(valid(m))
                def _():
                    for i in range(hb // _LANES):
                        ridx_s[half][pl.ds(i * _LANES, _LANES)] = (
                            srb[bi][pl.ds(_B + half * hb + i * _LANES,
                                          _LANES)])

                    @plsc.parallel_loop(half * hb, (half + 1) * hb, unroll=2)
                    def _edge(ei):
                        h0, h1 = plsc.unpack(
                            hv[gi][ei, :],
                            format=plsc.PackFormat.INTERLEAVED)
                        a = av[bi][pl.ds(ei * _NUM_LM, _LANES)]
                        t = []
                        for l in range(_NUM_L):
                            t.append((rv[bi][l, ei, pl.ds(0, _LANES)] * h0,
                                      rv[bi][l, ei,
                                             pl.ds(_LANES, _LANES)] * h1))
                        mvb = mv[half]
                        for lm in range(_NUM_LM):
                            tl = t[_LM_TO_L[lm]]
                            mvb[ei - half * hb, lm, :] = plsc.pack(
                                a[lm] * tl[0], a[lm] * tl[1],
                                format=plsc.PackFormat.INTERLEAVED)

                    pltpu.async_copy(mv[half], acc.at[ridx_s[half]],
                                     sem_sc[half], add=True)

        @pl.loop(0, k_chunks)
        def _chunk(k):
            pltpu.sync_copy(
                h_hbm.at[pl.ds(row0, rows_per_tile), pl.ds(k * _CL, _CL)],
                h_sh.at[pl.ds(row0, rows_per_tile)])
            pltpu.sync_copy(z_hbm, mv[0])
            hb = _B // 2
            for t in range(rows_per_tile // hb):
                pltpu.sync_copy(mv[0].at[pl.ds(0, hb)],
                                acc.at[pl.ds(row0 + t * hb, hb)])
            rem = rows_per_tile - (rows_per_tile // hb) * hb
            if rem:
                pltpu.sync_copy(mv[0].at[pl.ds(0, rem)],
                                acc.at[pl.ds(row0 + rows_per_tile - rem,
                                             rem)])
            plsc.subcore_barrier()

            for m in range(_NIN):
                issue_in(m, m % _NIN, k)
            wait_in(0, 0, k)
            issue_gather(0, 0, 0, k)

            @pl.loop(0, nslots // (_NIN * _NSC))
            def _slotgrp(jo):
                for t in range(_NIN * _NSC):
                    j = jo * (_NIN * _NSC) + t
                    bi, bi1 = t % _NIN, (t + 1) % _NIN
                    gi, gi1 = t % _NSC, (t + 1) % _NSC
                    wait_in(j + 1, bi1, k)
                    issue_gather(j + 1, bi1, gi1, k)
                    wait_gather(j, gi)
                    compute_and_scatter(j, bi, gi)
                    issue_in(j + _NIN, bi, k)

            for half in range(2):
                wait_scatter(nslots - 1, half)

            plsc.subcore_barrier()
            pltpu.sync_copy(
                acc.at[pl.ds(row0, rows_per_tile)],
                out_hbm.at[cid, pl.ds(row0, rows_per_tile), :,
                           pl.ds(k * _CL, _CL)])

    return sc_msg


def kernel(node_attrs, node_feats, edge_attrs, edge_feats, edge_index,
           W_up, W_skip, W_msg):
    n, c = node_feats.shape
    ne = node_attrs.shape[1]
    e = edge_attrs.shape[0]

    inv_sqrt_c = 1.0 / np.sqrt(c)
    w_up_s = W_up * inv_sqrt_c
    w_skip_flat = W_skip.reshape(c, ne * c) * (1.0 / np.sqrt(c * ne))
    w_msg_e = (W_msg[jnp.asarray(_LM_TO_L)] *
               (inv_sqrt_c / _AVG_NUM_NEIGHBORS))
    p = np.arange(c)
    perm = (p // _CL) * _CL + (p % _CL) // 2 + 16 * (p % 2)
    w_msg_e = w_msg_e[:, jnp.asarray(perm), :]

    h, sc = _dense_pre(node_feats, node_attrs, w_up_s, w_skip_flat)

    h2 = h[:, jnp.asarray(perm)].astype(jnp.bfloat16)
    eidx_blocked = jnp.concatenate(
        [edge_index[0].reshape(e // _B, _B),
         edge_index[1].reshape(e // _B, _B)], axis=1)
    zeros = jnp.zeros((_B // 2, _NUM_LM, _CL), jnp.bfloat16)

    ea_flat = edge_attrs.reshape(e * _NUM_LM)
    sc_fn = _make_sc_kernel(n, e, c)
    partials = sc_fn(h2, edge_feats, ea_flat, eidx_blocked, zeros)

    out = _dense_post(partials.reshape(_NC, n * _NUM_LM, c), w_msg_e)
    return (out, sc)


# speedup vs baseline: 1.1657x; 1.1657x over previous
import functools

import jax
import jax.numpy as jnp
import numpy as np
from jax import lax
from jax.experimental import pallas as pl
from jax.experimental.pallas import tpu as pltpu
from jax.experimental.pallas import tpu_sc as plsc

_NUM_L = 3
_NUM_LM = 9
_AVG_NUM_NEIGHBORS = 16.0
_LM_TO_L = np.repeat(np.arange(_NUM_L), [2 * l + 1 for l in range(_NUM_L)])

_B = 64
_LANES = 16
_CL = 32
_NC = 2
_NS = 16
_NIN = 2
_NSC = 2


def _dense_pre(node_feats, node_attrs, w_up_s, w_skip_flat):
    n, c = node_feats.shape
    ne = node_attrs.shape[1]
    blk = 1000
    grid = n // blk

    def body(nf_ref, na_ref, wup_ref, wsk_ref, h_ref, sc_ref):
        nf = nf_ref[...]
        h_ref[...] = jnp.dot(nf, wup_ref[...], preferred_element_type=jnp.float32)
        t = jnp.dot(nf, wsk_ref[...], preferred_element_type=jnp.float32)
        na = na_ref[...]
        sc_ref[...] = jnp.sum(t.reshape(blk, ne, c) * na[:, :, None], axis=1)

    return pl.pallas_call(
        body,
        grid=(grid,),
        in_specs=[
            pl.BlockSpec((blk, c), lambda i: (i, 0)),
            pl.BlockSpec((blk, ne), lambda i: (i, 0)),
            pl.BlockSpec((c, c), lambda i: (0, 0)),
            pl.BlockSpec((c, ne * c), lambda i: (0, 0)),
        ],
        out_specs=[
            pl.BlockSpec((blk, c), lambda i: (i, 0)),
            pl.BlockSpec((blk, c), lambda i: (i, 0)),
        ],
        out_shape=[
            jax.ShapeDtypeStruct((n, c), jnp.float32),
            jax.ShapeDtypeStruct((n, c), jnp.float32),
        ],
    )(node_feats, node_attrs, w_up_s, w_skip_flat)


def _dense_post(partials, w_msg_e):
    _, _, n, c = partials.shape
    blk = 1000
    grid = n // blk

    def body(p_ref, w_ref, o_ref):
        msg = (p_ref[0].astype(jnp.float32) +
               p_ref[1].astype(jnp.float32))
        for lm in range(_NUM_LM):
            o_ref[:, lm, :] = jnp.dot(msg[lm], w_ref[lm],
                                      preferred_element_type=jnp.float32)

    return pl.pallas_call(
        body,
        grid=(grid,),
        in_specs=[
            pl.BlockSpec((2, _NUM_LM, blk, c), lambda i: (0, 0, i, 0)),
            pl.BlockSpec((_NUM_LM, c, c), lambda i: (0, 0, 0)),
        ],
        out_specs=pl.BlockSpec((blk, _NUM_LM, c), lambda i: (i, 0, 0)),
        out_shape=jax.ShapeDtypeStruct((n, _NUM_LM, c), jnp.float32),
    )(partials, w_msg_e)


def _make_sc_kernel(n, e, c):
    k_chunks = c // _CL
    blocks = e // _B
    blocks_per_core = blocks // _NC
    rows_per_tile = n // _NS
    nslots = 80

    mesh = plsc.VectorSubcoreMesh(core_axis_name="c", subcore_axis_name="s")

    scratch = [pltpu.VMEM_SHARED((n, _NUM_LM, _CL), jnp.bfloat16)]
    scratch += [pltpu.VMEM_SHARED((n, _CL), jnp.bfloat16)]
    scratch += [pltpu.VMEM((2 * _B,), jnp.int32) for _ in range(_NIN)]
    scratch += [pltpu.VMEM((_B * _NUM_LM + _LANES,), jnp.float32)
                for _ in range(_NIN)]
    scratch += [pltpu.VMEM((_NUM_L, _B, _CL), jnp.float32)
                for _ in range(_NIN)]
    scratch += [pltpu.VMEM((_B,), jnp.int32) for _ in range(_NSC)]
    scratch += [pltpu.VMEM((_B, _CL), jnp.bfloat16) for _ in range(_NSC)]
    scratch += [pltpu.VMEM((_B // 2,), jnp.int32) for _ in range(_NSC)]
    scratch += [pltpu.VMEM((_B // 2, _NUM_LM, _CL), jnp.bfloat16)
                for _ in range(_NSC)]
    scratch += [pltpu.SemaphoreType.DMA for _ in range(_NIN)]
    scratch += [pltpu.SemaphoreType.DMA for _ in range(_NSC)]
    scratch += [pltpu.SemaphoreType.DMA for _ in range(_NSC)]

    @functools.partial(
        pl.kernel,
        out_type=jax.ShapeDtypeStruct((_NC, _NUM_LM, n, c), jnp.bfloat16),
        mesh=mesh,
        compiler_params=pltpu.CompilerParams(use_tc_tiling_on_sc=False,
                                             needs_layout_passes=False),
        scratch_types=scratch,
    )
    def sc_msg(h_hbm, ef_hbm, ea_hbm, sr_hbm, z_hbm, out_hbm, acc, h_sh,
               *sc):
        srb = sc[0:_NIN]
        av = sc[_NIN:2 * _NIN]
        rv = sc[2 * _NIN:3 * _NIN]
        o = 3 * _NIN
        gidx = sc[o:o + _NSC]
        hv = sc[o + _NSC:o + 2 * _NSC]
        ridx_s = sc[o + 2 * _NSC:o + 3 * _NSC]
        mv = sc[o + 3 * _NSC:o + 4 * _NSC]
        o = o + 4 * _NSC
        sem_in = sc[o:o + _NIN]
        sem_g = sc[o + _NIN:o + _NIN + _NSC]
        sem_sc = sc[o + _NIN + _NSC:o + _NIN + 2 * _NSC]

        cid = lax.axis_index("c")
        sid = lax.axis_index("s")
        row0 = sid * rows_per_tile

        def blk_of(m):
            return cid * blocks_per_core + sid + _NS * m

        def base_of(m):
            return blk_of(m) * _B

        def valid(m):
            return sid + _NS * m < blocks_per_core

        def in_descs(m, bi, k):
            b = base_of(m)
            return [
                (sr_hbm.at[blk_of(m), :], srb[bi]),
                (ea_hbm.at[pl.ds(b * _NUM_LM, _B * _NUM_LM)],
                 av[bi].at[pl.ds(0, _B * _NUM_LM)]),
            ] + [
                (ef_hbm.at[pl.ds(b, _B),
                           pl.ds(l * (k_chunks * _CL) + k * _CL, _CL)],
                 rv[bi].at[l])
                for l in range(_NUM_L)
            ]

        def issue_in(m, bi, k):
            @pl.when(valid(m))
            def _():
                for s, d in in_descs(m, bi, k):
                    pltpu.async_copy(s, d, sem_in[bi])

        def wait_in(m, bi, k):
            @pl.when(valid(m))
            def _():
                for s, d in in_descs(m, bi, k):
                    pltpu.make_async_copy(s, d, sem_in[bi]).wait()

        def issue_gather(m, bi, gi, k):
            @pl.when(valid(m))
            def _():
                for i in range(_B // _LANES):
                    gidx[gi][pl.ds(i * _LANES, _LANES)] = (
                        srb[bi][pl.ds(i * _LANES, _LANES)])
                pltpu.async_copy(h_sh.at[gidx[gi]], hv[gi], sem_g[gi])

        def wait_gather(m, gi):
            @pl.when(valid(m))
            def _():
                pltpu.make_async_copy(h_sh.at[gidx[gi]], hv[gi],
                                      sem_g[gi]).wait()

        def wait_scatter(m, half):
            @pl.when(valid(m) & (m >= 0))
            def _():
                pltpu.make_async_copy(mv[half], acc.at[ridx_s[half]],
                                      sem_sc[half]).wait()

        def compute_and_scatter(m, bi, gi):
            hb = _B // 2
            for half in range(2):
                wait_scatter(m - 1, half)

                @pl.when(valid(m))
                def _():
                    for i in range(hb // _LANES):
                        ridx_s[half][pl.ds(i * _LANES, _LANES)] = (
                            srb[bi][pl.ds(_B + half * hb + i * _LANES,
                                          _LANES)])

                    @plsc.parallel_loop(half * hb, (half + 1) * hb, unroll=2)
                    def _edge(ei):
                        h0, h1 = plsc.unpack(
                            hv[gi][ei, :],
                            format=plsc.PackFormat.INTERLEAVED)
                        a = av[bi][pl.ds(ei * _NUM_LM, _LANES)]
                        t = []
                        for l in range(_NUM_L):
                            t.append((rv[bi][l, ei, pl.ds(0, _LANES)] * h0,
                                      rv[bi][l, ei,
                                             pl.ds(_LANES, _LANES)] * h1))
                        mvb = mv[half]
                        for lm in range(_NUM_LM):
                            tl = t[_LM_TO_L[lm]]
                            mvb[ei - half * hb, lm, :] = plsc.pack(
                                a[lm] * tl[0], a[lm] * tl[1],
                                format=plsc.PackFormat.INTERLEAVED)

                    pltpu.async_copy(mv[half], acc.at[ridx_s[half]],
                                     sem_sc[half], add=True)

        @pl.loop(0, k_chunks)
        def _chunk(k):
            pltpu.sync_copy(
                h_hbm.at[pl.ds(row0, rows_per_tile), pl.ds(k * _CL, _CL)],
                h_sh.at[pl.ds(row0, rows_per_tile)])
            pltpu.sync_copy(z_hbm, mv[0])
            hb = _B // 2
            for t in range(rows_per_tile // hb):
                pltpu.sync_copy(mv[0].at[pl.ds(0, hb)],
                                acc.at[pl.ds(row0 + t * hb, hb)])
            rem = rows_per_tile - (rows_per_tile // hb) * hb
            if rem:
                pltpu.sync_copy(mv[0].at[pl.ds(0, rem)],
                                acc.at[pl.ds(row0 + rows_per_tile - rem,
                                             rem)])
            plsc.subcore_barrier()

            for m in range(_NIN):
                issue_in(m, m % _NIN, k)
            wait_in(0, 0, k)
            issue_gather(0, 0, 0, k)

            @pl.loop(0, nslots // (_NIN * _NSC))
            def _slotgrp(jo):
                for t in range(_NIN * _NSC):
                    j = jo * (_NIN * _NSC) + t
                    bi, bi1 = t % _NIN, (t + 1) % _NIN
                    gi, gi1 = t % _NSC, (t + 1) % _NSC
                    wait_in(j + 1, bi1, k)
                    issue_gather(j + 1, bi1, gi1, k)
                    wait_gather(j, gi)
                    compute_and_scatter(j, bi, gi)
                    issue_in(j + _NIN, bi, k)

            for half in range(2):
                wait_scatter(nslots - 1, half)

            plsc.subcore_barrier()
            for lm in range(_NUM_LM):
                pltpu.sync_copy(
                    acc.at[pl.ds(row0, rows_per_tile), lm, :],
                    out_hbm.at[cid, lm, pl.ds(row0, rows_per_tile),
                               pl.ds(k * _CL, _CL)])

    return sc_msg


def kernel(node_attrs, node_feats, edge_attrs, edge_feats, edge_index,
           W_up, W_skip, W_msg):
    n, c = node_feats.shape
    ne = node_attrs.shape[1]
    e = edge_attrs.shape[0]

    inv_sqrt_c = 1.0 / np.sqrt(c)
    w_up_s = W_up * inv_sqrt_c
    w_skip_flat = W_skip.reshape(c, ne * c) * (1.0 / np.sqrt(c * ne))
    w_msg_e = (W_msg[jnp.asarray(_LM_TO_L)] *
               (inv_sqrt_c / _AVG_NUM_NEIGHBORS))
    p = np.arange(c)
    perm = (p // _CL) * _CL + (p % _CL) // 2 + 16 * (p % 2)
    w_msg_e = w_msg_e[:, jnp.asarray(perm), :]

    h, sc = _dense_pre(node_feats, node_attrs, w_up_s, w_skip_flat)

    h2 = h[:, jnp.asarray(perm)].astype(jnp.bfloat16)
    eidx_blocked = jnp.concatenate(
        [edge_index[0].reshape(e // _B, _B),
         edge_index[1].reshape(e // _B, _B)], axis=1)
    zeros = jnp.zeros((_B // 2, _NUM_LM, _CL), jnp.bfloat16)

    ea_flat = edge_attrs.reshape(e * _NUM_LM)
    sc_fn = _make_sc_kernel(n, e, c)
    partials = sc_fn(h2, edge_feats, ea_flat, eidx_blocked, zeros)

    out = _dense_post(partials, w_msg_e)
    return (out, sc)
